# baseline (speedup 1.0000x reference)
import jax
import jax.numpy as jnp
from jax import lax
from jax.experimental import pallas as pl
from jax.experimental.pallas import tpu as pltpu
from jax.experimental.pallas import tpu_sc as plsc

B = 4096
K = 32

_info = plsc.get_sparse_core_info()
_NC = _info.num_cores
_NS = _info.num_subcores
_NW = _NC * _NS
_BPW = B // _NW


def _item_gather_body(item_hbm, itab_hbm, iout_hbm, iidx_v, irows_v, sem):
  wid = lax.axis_index("s") * _NC + lax.axis_index("c")
  base = wid * _BPW
  pltpu.sync_copy(item_hbm.at[pl.ds(base, _BPW)], iidx_v)
  pltpu.async_copy(itab_hbm.at[iidx_v], irows_v, sem).wait()
  pltpu.sync_copy(irows_v, iout_hbm.at[pl.ds(base, _BPW)])


_item_gather = pl.kernel(
    _item_gather_body,
    out_type=jax.ShapeDtypeStruct((B, K), jnp.float32),
    mesh=plsc.VectorSubcoreMesh(core_axis_name="c", subcore_axis_name="s"),
    scratch_types=[
        pltpu.VMEM((_BPW,), jnp.int32),
        pltpu.VMEM((_BPW, K), jnp.float32),
        pltpu.SemaphoreType.DMA,
    ],
    compiler_params=pltpu.CompilerParams(use_tc_tiling_on_sc=False),
)


_BM = 512
_NT = B // _BM


def _residual_body(rating_ref, u_ref, v_ref, out_ref):
  preds = lax.dot_general(
      u_ref[...], v_ref[...],
      dimension_numbers=(((1,), (1,)), ((), ())),
      preferred_element_type=jnp.float32)
  out_ref[...] = rating_ref[...] - preds


def _residual(rating, u_emb, i_emb):
  return pl.pallas_call(
      _residual_body,
      grid=(_NT,),
      in_specs=[
          pl.BlockSpec((_BM, B), lambda i: (i, 0)),
          pl.BlockSpec((_BM, K), lambda i: (i, 0)),
          pl.BlockSpec((B, K), lambda i: (0, 0)),
      ],
      out_specs=pl.BlockSpec((_BM, B), lambda i: (i, 0)),
      out_shape=jax.ShapeDtypeStruct((B, B), jnp.float32),
  )(rating, u_emb, i_emb)


@jax.jit
def kernel(user, item, rating, user_table, item_table):
  i_emb = _item_gather(item.astype(jnp.int32), item_table)
  u_emb = user_table.at[user].get(mode="promise_in_bounds")
  return _residual(rating, u_emb, i_emb)
